# baseline (speedup 1.0000x reference)
import numpy as np
import jax
import jax.numpy as jnp
from jax import lax
from jax.experimental import pallas as pl
from jax.experimental.pallas import tpu as pltpu
from jax.experimental.pallas import tpu_sc as plsc

_D = 256
_BN = 1280
_N = 160000
_NB = _N // _BN
_S = 10000
_SPAD = 11392
_BM = 712
_BW = _BN + 8
_WF = 128
_NR = _BN // 128

_TRI = np.triu(np.ones((128, 128), np.float32))
_TRIS = np.tril(np.ones((16, 16), np.float32), -1)


def _ranks(ix, lane, tri_ref, tris_ref, carry_ref, k):
    first = jnp.sum(jnp.where(lane == 0, ix, 0))
    last = jnp.sum(jnp.where(lane == _BN - 1, ix, 0))

    @pl.when(k == 0)
    def _():
        carry_ref[0] = 0
        carry_ref[1] = first

    r_prev = carry_ref[0]
    prev_last = carry_ref[1]
    shifted = jnp.concatenate([jnp.full((1, 1), prev_last, ix.dtype), ix[:, :-1]], axis=1)
    ball = (ix != shifted).astype(jnp.int32)
    b0 = jnp.sum(jnp.where(lane == 0, ball, 0))
    total = jnp.sum(ball)
    ball2 = jnp.concatenate(
        [ball.astype(jnp.bfloat16).reshape(_NR, 128),
         jnp.zeros((16 - _NR, 128), jnp.bfloat16)], axis=0)
    csum2 = jnp.dot(ball2, tri_ref[...], preferred_element_type=jnp.float32)
    t = jnp.dot(tris_ref[...], ball2, preferred_element_type=jnp.float32)
    pre = jnp.sum(t, axis=1, keepdims=True)
    csum = (csum2 + pre)[:_NR].reshape(1, _BN)
    r_first = r_prev + b0
    base = pl.multiple_of((r_first // 8) * 8, 8)
    c_row = csum.astype(jnp.int32) + (r_first - base - b0)
    nwin = r_first - base + total + 1
    carry_ref[0] = r_prev + total
    carry_ref[1] = last
    return base, c_row, nwin


def _onehot_t(c_row, w):
    iota_w = jax.lax.broadcasted_iota(jnp.int32, (w, _BN), 0)
    return (iota_w == c_row).astype(jnp.bfloat16)


def _seg_accum_kernel(ix_ref, x_ref, wg_ref, tri_ref, tris_ref, accx_ref,
                      accd_ref, r_ref, carry_ref):
    k = pl.program_id(0)

    @pl.when(k == 0)
    def _():
        accx_ref[...] = jnp.zeros_like(accx_ref)
        accd_ref[...] = jnp.zeros_like(accd_ref)

    ix = ix_ref[0]
    lane = jax.lax.broadcasted_iota(jnp.int32, (1, _BN), 1)
    base, c_row, nwin = _ranks(ix, lane, tri_ref, tris_ref, carry_ref, k)
    r_ref[0] = c_row + base

    x_bf = x_ref[...].astype(jnp.bfloat16)
    logit = jnp.dot(x_bf, wg_ref[...],
                    preferred_element_type=jnp.float32)
    e128 = jnp.exp(logit)
    e128b = e128.astype(jnp.bfloat16)
    vals = x_bf * e128b[:, 0:1]

    @pl.when(nwin <= _WF)
    def _():
        oh = _onehot_t(c_row, _WF)
        accx_ref[pl.ds(base, _WF), :] += jnp.dot(oh, vals, preferred_element_type=jnp.float32)
        accd_ref[pl.ds(base, _WF), :] += jnp.dot(oh, e128b, preferred_element_type=jnp.float32)

    @pl.when(nwin > _WF)
    def _():
        oh = _onehot_t(c_row, _BW)
        accx_ref[pl.ds(base, _BW), :] += jnp.dot(oh, vals, preferred_element_type=jnp.float32)
        accd_ref[pl.ds(base, _BW), :] += jnp.dot(oh, e128b, preferred_element_type=jnp.float32)


def _mlp_kernel(accx_ref, accd_ref, wf_ref, bf_ref, wh_ref, bh_ref, hy_ref):
    d = accd_ref[:, 0:1]
    t = accx_ref[...] / jnp.where(d > 0, d, 1.0)
    y = jax.lax.dot_general(t, wf_ref[...], (((1,), (1,)), ((), ())),
                            preferred_element_type=jnp.float32) + bf_ref[...]
    hy_ref[...] = jax.lax.dot_general(y, wh_ref[...], (((1,), (1,)), ((), ())),
                                      preferred_element_type=jnp.float32) + bh_ref[...]


_NW = 32
_RPW = _N // _NW
_CH = 128
_TAIL = _RPW - 39 * _CH


def _sc_expand_body(hy_hbm, r_hbm, out_hbm, idx_v, rows_v, idx8_v, rows8_v, sem):
    wid = lax.axis_index("s") * 2 + lax.axis_index("c")
    base = pl.multiple_of(wid * _RPW, 8)

    def chunk(off, idx, rows, n):
        pltpu.sync_copy(r_hbm.at[pl.ds(off, n)], idx)
        pltpu.async_copy(hy_hbm.at[idx], rows, sem).wait()
        pltpu.sync_copy(rows, out_hbm.at[pl.ds(off, n)])

    def outer(i, carry):
        for b in range(8):
            chunk(base + (i * 8 + b) * _CH, idx_v, rows_v, _CH)
        return carry

    lax.fori_loop(0, 4, outer, 0)
    for b in range(7):
        chunk(base + (32 + b) * _CH, idx_v, rows_v, _CH)
    chunk(base + 39 * _CH, idx8_v, rows8_v, _TAIL)


def _sc_expand(hy, rflat):
    mesh = plsc.VectorSubcoreMesh(core_axis_name="c", subcore_axis_name="s")
    f = pl.kernel(
        _sc_expand_body,
        mesh=mesh,
        out_type=jax.ShapeDtypeStruct((_N, _D), jnp.float32),
        scratch_types=[
            pltpu.VMEM((_CH,), jnp.int32),
            pltpu.VMEM((_CH, _D), jnp.float32),
            pltpu.VMEM((_TAIL,), jnp.int32),
            pltpu.VMEM((_TAIL, _D), jnp.float32),
            pltpu.SemaphoreType.DMA,
        ],
    )
    return f(hy, rflat)


def _expand_kernel(r_ref, hy_ref, out_ref):
    r = r_ref[0]
    lane = jax.lax.broadcasted_iota(jnp.int32, (1, _BN), 1)
    r0 = jnp.sum(jnp.where(lane == 0, r, 0))
    rlast = jnp.sum(jnp.where(lane == _BN - 1, r, 0))
    base = pl.multiple_of((r0 // 8) * 8, 8)
    c_row = r - base
    nwin = rlast - base + 1

    @pl.when(nwin <= _WF)
    def _():
        win = hy_ref[pl.ds(base, _WF), :].astype(jnp.bfloat16)
        out_ref[...] = jax.lax.dot_general(
            _onehot_t(c_row, _WF), win, (((0,), (0,)), ((), ())),
            preferred_element_type=jnp.float32)

    @pl.when(nwin > _WF)
    def _():
        win = hy_ref[pl.ds(base, _BW), :].astype(jnp.bfloat16)
        out_ref[...] = jax.lax.dot_general(
            _onehot_t(c_row, _BW), win, (((0,), (0,)), ((), ())),
            preferred_element_type=jnp.float32)


def kernel(x, ix, Wf, bf, Wg, bg, Wh, bh):
    x2 = x.reshape(_N, _D)
    ix3 = ix.astype(jnp.int32).reshape(_NB, 1, _BN)
    wg128 = jnp.broadcast_to(Wg.reshape(_D, 1), (_D, 128)).astype(jnp.bfloat16)
    tri = jnp.asarray(_TRI).astype(jnp.bfloat16)
    tris = jnp.asarray(_TRIS).astype(jnp.bfloat16)

    accx, accd, rks = pl.pallas_call(
        _seg_accum_kernel,
        grid=(_NB,),
        in_specs=[
            pl.BlockSpec((1, 1, _BN), lambda k: (k, 0, 0)),
            pl.BlockSpec((_BN, _D), lambda k: (k, 0)),
            pl.BlockSpec((_D, 128), lambda k: (0, 0)),
            pl.BlockSpec((128, 128), lambda k: (0, 0)),
            pl.BlockSpec((16, 16), lambda k: (0, 0)),
        ],
        out_specs=[
            pl.BlockSpec((_SPAD, _D), lambda k: (0, 0)),
            pl.BlockSpec((_SPAD, 128), lambda k: (0, 0)),
            pl.BlockSpec((1, 1, _BN), lambda k: (k, 0, 0)),
        ],
        out_shape=[
            jax.ShapeDtypeStruct((_SPAD, _D), jnp.float32),
            jax.ShapeDtypeStruct((_SPAD, 128), jnp.float32),
            jax.ShapeDtypeStruct((_NB, 1, _BN), jnp.int32),
        ],
        scratch_shapes=[pltpu.SMEM((2,), jnp.int32)],
    )(ix3, x2, wg128, tri, tris)

    hy = pl.pallas_call(
        _mlp_kernel,
        grid=(_SPAD // _BM,),
        in_specs=[
            pl.BlockSpec((_BM, _D), lambda k: (k, 0)),
            pl.BlockSpec((_BM, 128), lambda k: (k, 0)),
            pl.BlockSpec((_D, _D), lambda k: (0, 0)),
            pl.BlockSpec((1, _D), lambda k: (0, 0)),
            pl.BlockSpec((_D, _D), lambda k: (0, 0)),
            pl.BlockSpec((1, _D), lambda k: (0, 0)),
        ],
        out_specs=pl.BlockSpec((_BM, _D), lambda k: (k, 0)),
        out_shape=jax.ShapeDtypeStruct((_SPAD, _D), jnp.float32),
    )(accx, accd, Wf, bf.reshape(1, _D), Wh, bh.reshape(1, _D))

    out = _sc_expand(hy, rks.reshape(_N))

    return out.reshape(1, _N, _D)


# speedup vs baseline: 1.1487x; 1.1487x over previous
import numpy as np
import jax
import jax.numpy as jnp
from jax import lax
from jax.experimental import pallas as pl
from jax.experimental.pallas import tpu as pltpu
from jax.experimental.pallas import tpu_sc as plsc

_D = 256
_BN = 1280
_N = 160000
_NB = _N // _BN
_S = 10000
_SPAD = 11392
_BM = 712
_BW = _BN + 8
_WF = 128
_NR = _BN // 128

_TRI = np.triu(np.ones((128, 128), np.float32))
_TRIS = np.tril(np.ones((16, 16), np.float32), -1)


def _ranks(ix, lane, tri_ref, tris_ref, carry_ref, k):
    first = jnp.sum(jnp.where(lane == 0, ix, 0))
    last = jnp.sum(jnp.where(lane == _BN - 1, ix, 0))

    @pl.when(k == 0)
    def _():
        carry_ref[0] = 0
        carry_ref[1] = first

    r_prev = carry_ref[0]
    prev_last = carry_ref[1]
    shifted = jnp.concatenate([jnp.full((1, 1), prev_last, ix.dtype), ix[:, :-1]], axis=1)
    ball = (ix != shifted).astype(jnp.int32)
    b0 = jnp.sum(jnp.where(lane == 0, ball, 0))
    total = jnp.sum(ball)
    ball2 = jnp.concatenate(
        [ball.astype(jnp.bfloat16).reshape(_NR, 128),
         jnp.zeros((16 - _NR, 128), jnp.bfloat16)], axis=0)
    csum2 = jnp.dot(ball2, tri_ref[...], preferred_element_type=jnp.float32)
    t = jnp.dot(tris_ref[...], ball2, preferred_element_type=jnp.float32)
    pre = jnp.sum(t, axis=1, keepdims=True)
    csum = (csum2 + pre)[:_NR].reshape(1, _BN)
    r_first = r_prev + b0
    base = pl.multiple_of((r_first // 8) * 8, 8)
    c_row = csum.astype(jnp.int32) + (r_first - base - b0)
    nwin = r_first - base + total + 1
    carry_ref[0] = r_prev + total
    carry_ref[1] = last
    return base, c_row, nwin


def _onehot_t(c_row, w):
    iota_w = jax.lax.broadcasted_iota(jnp.int32, (w, _BN), 0)
    return (iota_w == c_row).astype(jnp.bfloat16)


def _seg_accum_kernel(ix_ref, x_ref, wg_ref, tri_ref, tris_ref, accx_ref,
                      accd_ref, r_ref, carry_ref):
    k = pl.program_id(0)

    @pl.when(k == 0)
    def _():
        accx_ref[...] = jnp.zeros_like(accx_ref)
        accd_ref[...] = jnp.zeros_like(accd_ref)

    ix = ix_ref[0]
    lane = jax.lax.broadcasted_iota(jnp.int32, (1, _BN), 1)
    base, c_row, nwin = _ranks(ix, lane, tri_ref, tris_ref, carry_ref, k)
    r_ref[0] = c_row + base

    x_bf = x_ref[...].astype(jnp.bfloat16)
    logit = jnp.dot(x_bf, wg_ref[...],
                    preferred_element_type=jnp.float32)
    e128 = jnp.exp(logit)
    e128b = e128.astype(jnp.bfloat16)
    vals = x_bf * e128b[:, 0:1]

    @pl.when(nwin <= _WF)
    def _():
        oh = _onehot_t(c_row, _WF)
        accx_ref[pl.ds(base, _WF), :] += jnp.dot(oh, vals, preferred_element_type=jnp.float32)
        accd_ref[pl.ds(base, _WF), :] += jnp.dot(oh, e128b, preferred_element_type=jnp.float32)

    @pl.when(nwin > _WF)
    def _():
        oh = _onehot_t(c_row, _BW)
        accx_ref[pl.ds(base, _BW), :] += jnp.dot(oh, vals, preferred_element_type=jnp.float32)
        accd_ref[pl.ds(base, _BW), :] += jnp.dot(oh, e128b, preferred_element_type=jnp.float32)


def _mlp_kernel(accx_ref, accd_ref, wf_ref, bf_ref, wh_ref, bh_ref, hy_ref):
    d = accd_ref[:, 0:1]
    t = accx_ref[...] / jnp.where(d > 0, d, 1.0)
    y = jax.lax.dot_general(t, wf_ref[...], (((1,), (1,)), ((), ())),
                            preferred_element_type=jnp.float32) + bf_ref[...]
    hy_ref[...] = jax.lax.dot_general(y, wh_ref[...], (((1,), (1,)), ((), ())),
                                      preferred_element_type=jnp.float32) + bh_ref[...]


_NW = 32
_RPW = _N // _NW
_CH = 128
_TAIL = _RPW - 39 * _CH


def _sc_expand_body(hy_hbm, r_hbm, out_hbm, idx0, idx1, rows0, rows1,
                    idx8_v, rows8_v, sem0, sem1, sem8):
    wid = lax.axis_index("s") * 2 + lax.axis_index("c")
    base = pl.multiple_of(wid * _RPW, 8)
    idx = (idx0, idx1)
    rows = (rows0, rows1)
    sem = (sem0, sem1)

    def start(off, b):
        pltpu.sync_copy(r_hbm.at[pl.ds(off, _CH)], idx[b])
        return pltpu.async_copy(hy_hbm.at[idx[b]], rows[b], sem[b])

    def group(goff):
        cp = start(goff, 0)
        for b in range(8):
            nxt = start(goff + (b + 1) * _CH, (b + 1) % 2) if b + 1 < 8 else None
            cp.wait()
            pltpu.sync_copy(rows[b % 2], out_hbm.at[pl.ds(goff + b * _CH, _CH)])
            cp = nxt

    def outer(i, carry):
        group(base + i * 8 * _CH)
        return carry

    lax.fori_loop(0, 4, outer, 0)
    cp = start(base + 32 * _CH, 0)
    for b in range(7):
        nxt = start(base + (33 + b) * _CH, (b + 1) % 2) if b + 1 < 7 else None
        cp.wait()
        pltpu.sync_copy(rows[b % 2], out_hbm.at[pl.ds(base + (32 + b) * _CH, _CH)])
        cp = nxt
    off = base + 39 * _CH
    pltpu.sync_copy(r_hbm.at[pl.ds(off, _TAIL)], idx8_v)
    pltpu.async_copy(hy_hbm.at[idx8_v], rows8_v, sem8).wait()
    pltpu.sync_copy(rows8_v, out_hbm.at[pl.ds(off, _TAIL)])


def _sc_expand(hy, rflat):
    mesh = plsc.VectorSubcoreMesh(core_axis_name="c", subcore_axis_name="s")
    f = pl.kernel(
        _sc_expand_body,
        mesh=mesh,
        out_type=jax.ShapeDtypeStruct((_N, _D), jnp.float32),
        scratch_types=[
            pltpu.VMEM((_CH,), jnp.int32),
            pltpu.VMEM((_CH,), jnp.int32),
            pltpu.VMEM((_CH, _D), jnp.float32),
            pltpu.VMEM((_CH, _D), jnp.float32),
            pltpu.VMEM((_TAIL,), jnp.int32),
            pltpu.VMEM((_TAIL, _D), jnp.float32),
            pltpu.SemaphoreType.DMA,
            pltpu.SemaphoreType.DMA,
            pltpu.SemaphoreType.DMA,
        ],
    )
    return f(hy, rflat)


def _expand_kernel(r_ref, hy_ref, out_ref):
    r = r_ref[0]
    lane = jax.lax.broadcasted_iota(jnp.int32, (1, _BN), 1)
    r0 = jnp.sum(jnp.where(lane == 0, r, 0))
    rlast = jnp.sum(jnp.where(lane == _BN - 1, r, 0))
    base = pl.multiple_of((r0 // 8) * 8, 8)
    c_row = r - base
    nwin = rlast - base + 1

    @pl.when(nwin <= _WF)
    def _():
        win = hy_ref[pl.ds(base, _WF), :].astype(jnp.bfloat16)
        out_ref[...] = jax.lax.dot_general(
            _onehot_t(c_row, _WF), win, (((0,), (0,)), ((), ())),
            preferred_element_type=jnp.float32)

    @pl.when(nwin > _WF)
    def _():
        win = hy_ref[pl.ds(base, _BW), :].astype(jnp.bfloat16)
        out_ref[...] = jax.lax.dot_general(
            _onehot_t(c_row, _BW), win, (((0,), (0,)), ((), ())),
            preferred_element_type=jnp.float32)


def kernel(x, ix, Wf, bf, Wg, bg, Wh, bh):
    x2 = x.reshape(_N, _D)
    ix3 = ix.astype(jnp.int32).reshape(_NB, 1, _BN)
    wg128 = jnp.broadcast_to(Wg.reshape(_D, 1), (_D, 128)).astype(jnp.bfloat16)
    tri = jnp.asarray(_TRI).astype(jnp.bfloat16)
    tris = jnp.asarray(_TRIS).astype(jnp.bfloat16)

    accx, accd, rks = pl.pallas_call(
        _seg_accum_kernel,
        grid=(_NB,),
        in_specs=[
            pl.BlockSpec((1, 1, _BN), lambda k: (k, 0, 0)),
            pl.BlockSpec((_BN, _D), lambda k: (k, 0)),
            pl.BlockSpec((_D, 128), lambda k: (0, 0)),
            pl.BlockSpec((128, 128), lambda k: (0, 0)),
            pl.BlockSpec((16, 16), lambda k: (0, 0)),
        ],
        out_specs=[
            pl.BlockSpec((_SPAD, _D), lambda k: (0, 0)),
            pl.BlockSpec((_SPAD, 128), lambda k: (0, 0)),
            pl.BlockSpec((1, 1, _BN), lambda k: (k, 0, 0)),
        ],
        out_shape=[
            jax.ShapeDtypeStruct((_SPAD, _D), jnp.float32),
            jax.ShapeDtypeStruct((_SPAD, 128), jnp.float32),
            jax.ShapeDtypeStruct((_NB, 1, _BN), jnp.int32),
        ],
        scratch_shapes=[pltpu.SMEM((2,), jnp.int32)],
    )(ix3, x2, wg128, tri, tris)

    hy = pl.pallas_call(
        _mlp_kernel,
        grid=(_SPAD // _BM,),
        in_specs=[
            pl.BlockSpec((_BM, _D), lambda k: (k, 0)),
            pl.BlockSpec((_BM, 128), lambda k: (k, 0)),
            pl.BlockSpec((_D, _D), lambda k: (0, 0)),
            pl.BlockSpec((1, _D), lambda k: (0, 0)),
            pl.BlockSpec((_D, _D), lambda k: (0, 0)),
            pl.BlockSpec((1, _D), lambda k: (0, 0)),
        ],
        out_specs=pl.BlockSpec((_BM, _D), lambda k: (k, 0)),
        out_shape=jax.ShapeDtypeStruct((_SPAD, _D), jnp.float32),
    )(accx, accd, Wf, bf.reshape(1, _D), Wh, bh.reshape(1, _D))

    out = _sc_expand(hy, rks.reshape(_N))

    return out.reshape(1, _N, _D)


# speedup vs baseline: 1.7028x; 1.4824x over previous
import numpy as np
import jax
import jax.numpy as jnp
from jax import lax
from jax.experimental import pallas as pl
from jax.experimental.pallas import tpu as pltpu

_D = 256
_BN = 1280
_N = 160000
_NB = _N // _BN
_S = 10000
_SPAD = 11392
_BM = 712
_BW = _BN + 8
_WF = 128
_NR = _BN // 128

_TRI = np.triu(np.ones((128, 128), np.float32))
_TRIS = np.tril(np.ones((16, 16), np.float32), -1)


def _ranks(ix, lane, tri_ref, tris_ref, carry_ref, k):
    first = jnp.sum(jnp.where(lane == 0, ix, 0))
    last = jnp.sum(jnp.where(lane == _BN - 1, ix, 0))

    @pl.when(k == 0)
    def _():
        carry_ref[0] = 0
        carry_ref[1] = first

    r_prev = carry_ref[0]
    prev_last = carry_ref[1]
    shifted = jnp.concatenate([jnp.full((1, 1), prev_last, ix.dtype), ix[:, :-1]], axis=1)
    ball = (ix != shifted).astype(jnp.int32)
    b0 = jnp.sum(jnp.where(lane == 0, ball, 0))
    total = jnp.sum(ball)
    ball2 = jnp.concatenate(
        [ball.astype(jnp.bfloat16).reshape(_NR, 128),
         jnp.zeros((16 - _NR, 128), jnp.bfloat16)], axis=0)
    csum2 = jnp.dot(ball2, tri_ref[...], preferred_element_type=jnp.float32)
    t = jnp.dot(tris_ref[...], ball2, preferred_element_type=jnp.float32)
    pre = jnp.sum(t, axis=1, keepdims=True)
    csum = (csum2 + pre)[:_NR].reshape(1, _BN)
    r_first = r_prev + b0
    base = pl.multiple_of((r_first // 8) * 8, 8)
    c_row = csum.astype(jnp.int32) + (r_first - base - b0)
    nwin = r_first - base + total + 1
    carry_ref[0] = r_prev + total
    carry_ref[1] = last
    return base, c_row, nwin


def _onehot_t(c_row, w):
    iota_w = jax.lax.broadcasted_iota(jnp.int32, (w, _BN), 0)
    return (iota_w == c_row).astype(jnp.bfloat16)


def _seg_accum_kernel(ix_ref, x_ref, wg_ref, tri_ref, tris_ref, accx_ref,
                      accd_ref, r_ref, carry_ref):
    k = pl.program_id(0)

    @pl.when(k == 0)
    def _():
        accx_ref[...] = jnp.zeros_like(accx_ref)
        accd_ref[...] = jnp.zeros_like(accd_ref)

    ix = ix_ref[0]
    lane = jax.lax.broadcasted_iota(jnp.int32, (1, _BN), 1)
    base, c_row, nwin = _ranks(ix, lane, tri_ref, tris_ref, carry_ref, k)
    r_ref[0] = c_row + base

    x_bf = x_ref[...].astype(jnp.bfloat16)
    logit = jnp.dot(x_bf, wg_ref[...],
                    preferred_element_type=jnp.float32)
    e128 = jnp.exp(logit)
    e128b = e128.astype(jnp.bfloat16)
    vals = x_bf * e128b[:, 0:1]

    @pl.when(nwin <= _WF)
    def _():
        oh = _onehot_t(c_row, _WF)
        accx_ref[pl.ds(base, _WF), :] += jnp.dot(oh, vals, preferred_element_type=jnp.float32)
        accd_ref[pl.ds(base, _WF), :] += jnp.dot(oh, e128b, preferred_element_type=jnp.float32)

    @pl.when(nwin > _WF)
    def _():
        oh = _onehot_t(c_row, _BW)
        accx_ref[pl.ds(base, _BW), :] += jnp.dot(oh, vals, preferred_element_type=jnp.float32)
        accd_ref[pl.ds(base, _BW), :] += jnp.dot(oh, e128b, preferred_element_type=jnp.float32)


def _mlp_kernel(accx_ref, accd_ref, wf_ref, bf_ref, wh_ref, bh_ref, hy_ref):
    d = accd_ref[:, 0:1]
    t = accx_ref[...] / jnp.where(d > 0, d, 1.0)
    y = jax.lax.dot_general(t, wf_ref[...], (((1,), (1,)), ((), ())),
                            preferred_element_type=jnp.float32) + bf_ref[...]
    hy_ref[...] = jax.lax.dot_general(y, wh_ref[...], (((1,), (1,)), ((), ())),
                                      preferred_element_type=jnp.float32) + bh_ref[...]


def _expand_kernel(r_ref, hy_ref, out_ref):
    r = r_ref[0]
    lane = jax.lax.broadcasted_iota(jnp.int32, (1, _BN), 1)
    r0 = jnp.sum(jnp.where(lane == 0, r, 0))
    rlast = jnp.sum(jnp.where(lane == _BN - 1, r, 0))
    base = pl.multiple_of((r0 // 8) * 8, 8)
    c_row = r - base
    nwin = rlast - base + 1

    @pl.when(nwin <= _WF)
    def _():
        win = hy_ref[pl.ds(base, _WF), :].astype(jnp.bfloat16)
        out_ref[...] = jax.lax.dot_general(
            _onehot_t(c_row, _WF), win, (((0,), (0,)), ((), ())),
            preferred_element_type=jnp.float32)

    @pl.when(nwin > _WF)
    def _():
        win = hy_ref[pl.ds(base, _BW), :].astype(jnp.bfloat16)
        out_ref[...] = jax.lax.dot_general(
            _onehot_t(c_row, _BW), win, (((0,), (0,)), ((), ())),
            preferred_element_type=jnp.float32)


def kernel(x, ix, Wf, bf, Wg, bg, Wh, bh):
    x2 = x.reshape(_N, _D)
    ix3 = ix.astype(jnp.int32).reshape(_NB, 1, _BN)
    wg128 = jnp.broadcast_to(Wg.reshape(_D, 1), (_D, 128)).astype(jnp.bfloat16)
    tri = jnp.asarray(_TRI).astype(jnp.bfloat16)
    tris = jnp.asarray(_TRIS).astype(jnp.bfloat16)

    accx, accd, rks = pl.pallas_call(
        _seg_accum_kernel,
        grid=(_NB,),
        in_specs=[
            pl.BlockSpec((1, 1, _BN), lambda k: (k, 0, 0)),
            pl.BlockSpec((_BN, _D), lambda k: (k, 0)),
            pl.BlockSpec((_D, 128), lambda k: (0, 0)),
            pl.BlockSpec((128, 128), lambda k: (0, 0)),
            pl.BlockSpec((16, 16), lambda k: (0, 0)),
        ],
        out_specs=[
            pl.BlockSpec((_SPAD, _D), lambda k: (0, 0)),
            pl.BlockSpec((_SPAD, 128), lambda k: (0, 0)),
            pl.BlockSpec((1, 1, _BN), lambda k: (k, 0, 0)),
        ],
        out_shape=[
            jax.ShapeDtypeStruct((_SPAD, _D), jnp.float32),
            jax.ShapeDtypeStruct((_SPAD, 128), jnp.float32),
            jax.ShapeDtypeStruct((_NB, 1, _BN), jnp.int32),
        ],
        scratch_shapes=[pltpu.SMEM((2,), jnp.int32)],
    )(ix3, x2, wg128, tri, tris)

    hy = pl.pallas_call(
        _mlp_kernel,
        grid=(_SPAD // _BM,),
        in_specs=[
            pl.BlockSpec((_BM, _D), lambda k: (k, 0)),
            pl.BlockSpec((_BM, 128), lambda k: (k, 0)),
            pl.BlockSpec((_D, _D), lambda k: (0, 0)),
            pl.BlockSpec((1, _D), lambda k: (0, 0)),
            pl.BlockSpec((_D, _D), lambda k: (0, 0)),
            pl.BlockSpec((1, _D), lambda k: (0, 0)),
        ],
        out_specs=pl.BlockSpec((_BM, _D), lambda k: (k, 0)),
        out_shape=jax.ShapeDtypeStruct((_SPAD, _D), jnp.float32),
    )(accx, accd, Wf, bf.reshape(1, _D), Wh, bh.reshape(1, _D))

    out = pl.pallas_call(
        _expand_kernel,
        grid=(_NB,),
        in_specs=[
            pl.BlockSpec((1, 1, _BN), lambda k: (k, 0, 0)),
            pl.BlockSpec((_SPAD, _D), lambda k: (0, 0)),
        ],
        out_specs=pl.BlockSpec((_BN, _D), lambda k: (k, 0)),
        out_shape=jax.ShapeDtypeStruct((_N, _D), jnp.float32),
    )(rks, hy)

    return out.reshape(1, _N, _D)


# speedup vs baseline: 1.7188x; 1.0094x over previous
import numpy as np
import jax
import jax.numpy as jnp
from jax import lax
from jax.experimental import pallas as pl
from jax.experimental.pallas import tpu as pltpu

_D = 256
_BN = 1280
_N = 160000
_NB = _N // _BN
_S = 10000
_SPAD = 11392
_BM = 712
_BW = _BN + 8
_WF = 128
_NR = _BN // 128

_TRI = np.triu(np.ones((128, 128), np.float32))
_TRIS = np.tril(np.ones((16, 16), np.float32), -1)


def _ranks(ix, lane, tri_ref, tris_ref, carry_ref, k):
    first = jnp.sum(jnp.where(lane == 0, ix, 0))
    last = jnp.sum(jnp.where(lane == _BN - 1, ix, 0))

    @pl.when(k == 0)
    def _():
        carry_ref[0] = 0
        carry_ref[1] = first

    r_prev = carry_ref[0]
    prev_last = carry_ref[1]
    shifted = jnp.concatenate([jnp.full((1, 1), prev_last, ix.dtype), ix[:, :-1]], axis=1)
    ball = (ix != shifted).astype(jnp.int32)
    b0 = jnp.sum(jnp.where(lane == 0, ball, 0))
    total = jnp.sum(ball)
    ball2 = jnp.concatenate(
        [ball.astype(jnp.bfloat16).reshape(_NR, 128),
         jnp.zeros((16 - _NR, 128), jnp.bfloat16)], axis=0)
    csum2 = jnp.dot(ball2, tri_ref[...], preferred_element_type=jnp.float32)
    t = jnp.dot(tris_ref[...], ball2, preferred_element_type=jnp.float32)
    pre = jnp.sum(t, axis=1, keepdims=True)
    csum = (csum2 + pre)[:_NR].reshape(1, _BN)
    r_first = r_prev + b0
    base = pl.multiple_of((r_first // 8) * 8, 8)
    c_row = csum.astype(jnp.int32) + (r_first - base - b0)
    nwin = r_first - base + total + 1
    carry_ref[0] = r_prev + total
    carry_ref[1] = last
    return base, c_row, nwin


def _onehot_t(c_row, w):
    iota_w = jax.lax.broadcasted_iota(jnp.int32, (w, _BN), 0)
    return (iota_w == c_row).astype(jnp.bfloat16)


def _seg_accum_kernel(ix_ref, x_ref, wg_ref, tri_ref, tris_ref, accx_ref,
                      accd_ref, r_ref, carry_ref):
    k = pl.program_id(0)

    @pl.when(k == 0)
    def _():
        accx_ref[...] = jnp.zeros_like(accx_ref)
        accd_ref[...] = jnp.zeros_like(accd_ref)

    ix = ix_ref[0]
    lane = jax.lax.broadcasted_iota(jnp.int32, (1, _BN), 1)
    base, c_row, nwin = _ranks(ix, lane, tri_ref, tris_ref, carry_ref, k)
    r_ref[0] = c_row + base

    x_bf = x_ref[...].astype(jnp.bfloat16)
    logit = jnp.dot(x_bf, wg_ref[...],
                    preferred_element_type=jnp.float32)
    e128 = jnp.exp(logit)
    e128b = e128.astype(jnp.bfloat16)
    vals = x_bf * e128b[:, 0:1]

    @pl.when(nwin <= _WF)
    def _():
        oh = _onehot_t(c_row, _WF)
        accx_ref[pl.ds(base, _WF), :] += jnp.dot(oh, vals, preferred_element_type=jnp.float32)
        accd_ref[pl.ds(base, _WF), :] += jnp.dot(oh, e128b, preferred_element_type=jnp.float32)

    @pl.when(nwin > _WF)
    def _():
        oh = _onehot_t(c_row, _BW)
        accx_ref[pl.ds(base, _BW), :] += jnp.dot(oh, vals, preferred_element_type=jnp.float32)
        accd_ref[pl.ds(base, _BW), :] += jnp.dot(oh, e128b, preferred_element_type=jnp.float32)


def _mlp_kernel(accx_ref, accd_ref, wf_ref, bf_ref, wh_ref, bh_ref, hy_ref):
    d = accd_ref[:, 0:1]
    t = accx_ref[...] / jnp.where(d > 0, d, 1.0)
    y = jax.lax.dot_general(t, wf_ref[...], (((1,), (1,)), ((), ())),
                            preferred_element_type=jnp.float32) + bf_ref[...]
    hy = jax.lax.dot_general(y, wh_ref[...], (((1,), (1,)), ((), ())),
                             preferred_element_type=jnp.float32) + bh_ref[...]
    hy_ref[...] = hy.astype(jnp.bfloat16)


def _expand_kernel(r_ref, hy_ref, out_ref):
    r = r_ref[0]
    lane = jax.lax.broadcasted_iota(jnp.int32, (1, _BN), 1)
    r0 = jnp.sum(jnp.where(lane == 0, r, 0))
    rlast = jnp.sum(jnp.where(lane == _BN - 1, r, 0))
    base = pl.multiple_of((r0 // 8) * 8, 8)
    c_row = r - base
    nwin = rlast - base + 1

    @pl.when(nwin <= _WF)
    def _():
        win = hy_ref[pl.ds(base, _WF), :]
        out_ref[...] = jax.lax.dot_general(
            _onehot_t(c_row, _WF), win, (((0,), (0,)), ((), ())),
            preferred_element_type=jnp.float32)

    @pl.when(nwin > _WF)
    def _():
        win = hy_ref[pl.ds(base, _BW), :]
        out_ref[...] = jax.lax.dot_general(
            _onehot_t(c_row, _BW), win, (((0,), (0,)), ((), ())),
            preferred_element_type=jnp.float32)


def kernel(x, ix, Wf, bf, Wg, bg, Wh, bh):
    x2 = x.reshape(_N, _D)
    ix3 = ix.astype(jnp.int32).reshape(_NB, 1, _BN)
    wg128 = jnp.broadcast_to(Wg.reshape(_D, 1), (_D, 128)).astype(jnp.bfloat16)
    tri = jnp.asarray(_TRI).astype(jnp.bfloat16)
    tris = jnp.asarray(_TRIS).astype(jnp.bfloat16)

    accx, accd, rks = pl.pallas_call(
        _seg_accum_kernel,
        grid=(_NB,),
        in_specs=[
            pl.BlockSpec((1, 1, _BN), lambda k: (k, 0, 0)),
            pl.BlockSpec((_BN, _D), lambda k: (k, 0)),
            pl.BlockSpec((_D, 128), lambda k: (0, 0)),
            pl.BlockSpec((128, 128), lambda k: (0, 0)),
            pl.BlockSpec((16, 16), lambda k: (0, 0)),
        ],
        out_specs=[
            pl.BlockSpec((_SPAD, _D), lambda k: (0, 0)),
            pl.BlockSpec((_SPAD, 128), lambda k: (0, 0)),
            pl.BlockSpec((1, 1, _BN), lambda k: (k, 0, 0)),
        ],
        out_shape=[
            jax.ShapeDtypeStruct((_SPAD, _D), jnp.float32),
            jax.ShapeDtypeStruct((_SPAD, 128), jnp.float32),
            jax.ShapeDtypeStruct((_NB, 1, _BN), jnp.int32),
        ],
        scratch_shapes=[pltpu.SMEM((2,), jnp.int32)],
    )(ix3, x2, wg128, tri, tris)

    hy = pl.pallas_call(
        _mlp_kernel,
        grid=(_SPAD // _BM,),
        in_specs=[
            pl.BlockSpec((_BM, _D), lambda k: (k, 0)),
            pl.BlockSpec((_BM, 128), lambda k: (k, 0)),
            pl.BlockSpec((_D, _D), lambda k: (0, 0)),
            pl.BlockSpec((1, _D), lambda k: (0, 0)),
            pl.BlockSpec((_D, _D), lambda k: (0, 0)),
            pl.BlockSpec((1, _D), lambda k: (0, 0)),
        ],
        out_specs=pl.BlockSpec((_BM, _D), lambda k: (k, 0)),
        out_shape=jax.ShapeDtypeStruct((_SPAD, _D), jnp.bfloat16),
    )(accx, accd, Wf, bf.reshape(1, _D), Wh, bh.reshape(1, _D))

    out = pl.pallas_call(
        _expand_kernel,
        grid=(_NB,),
        in_specs=[
            pl.BlockSpec((1, 1, _BN), lambda k: (k, 0, 0)),
            pl.BlockSpec((_SPAD, _D), lambda k: (0, 0)),
        ],
        out_specs=pl.BlockSpec((_BN, _D), lambda k: (k, 0)),
        out_shape=jax.ShapeDtypeStruct((_N, _D), jnp.float32),
    )(rks, hy)

    return out.reshape(1, _N, _D)
